# baseline (speedup 1.0000x reference)
import functools

import jax
import jax.numpy as jnp
from jax import lax
from jax.experimental import pallas as pl
from jax.experimental.pallas import tpu as pltpu
from jax.experimental.pallas import tpu_sc as plsc

N = 10000
E = 320000
D = 128
ACT = 18
NC = 2
NS = 16
NT = NC * NS
CH = 128
NCH = 80
TCH = NT * NCH
CAP = TCH * CH
SCH = 40
NCH0 = 40
NCH1 = 120
BASE1 = NS * NCH0
NROWS = 10112
STRIPE = NROWS // NS
DUMMY = 10008
DEGW = 128

NBUF = 2
SUB = 4
SUBR = CH // SUB

_mesh = plsc.VectorSubcoreMesh(core_axis_name="c", subcore_axis_name="s")


@functools.partial(
    pl.kernel,
    out_type=jax.ShapeDtypeStruct((NC, NROWS, DEGW), jnp.float32),
    mesh=_mesh,
    scratch_types=[
        pltpu.VMEM((NCH, CH), jnp.int32),
        pltpu.VMEM((CH, DEGW), jnp.float32),
        pltpu.VMEM_SHARED((NROWS, DEGW), jnp.float32),
    ],
)
def _deg_kernel(dsts_hbm, ones_hbm, zeros_hbm, out_hbm, dst_v, ones_v, deg_sh):
    c = lax.axis_index("c")
    s = lax.axis_index("s")
    wid = c * NS + s
    pltpu.sync_copy(dsts_hbm.at[wid], dst_v)
    pltpu.sync_copy(ones_hbm, ones_v)
    pltpu.sync_copy(zeros_hbm.at[pl.ds(s * STRIPE, STRIPE)],
                    deg_sh.at[pl.ds(s * STRIPE, STRIPE)])
    plsc.subcore_barrier()

    def body(i, carry):
        pltpu.sync_copy(ones_v, deg_sh.at[dst_v.at[i]], add=True)
        return carry

    lax.fori_loop(0, NCH, body, 0)
    plsc.subcore_barrier()
    pltpu.sync_copy(deg_sh.at[pl.ds(s * STRIPE, STRIPE)],
                    out_hbm.at[c, pl.ds(s * STRIPE, STRIPE)])


@functools.partial(
    pl.kernel,
    out_type=jax.ShapeDtypeStruct((NC, NROWS, D), jnp.float32),
    mesh=_mesh,
    scratch_types=[
        pltpu.VMEM((SCH * CH,), jnp.int32),
        pltpu.VMEM((SCH, CH), jnp.int32),
        *[pltpu.VMEM((CH, D), jnp.float32) for _ in range(NBUF)],
        pltpu.VMEM_SHARED((NROWS, D), jnp.float32),
        *[pltpu.SemaphoreType.DMA for _ in range(NBUF)],
    ],
)
def _agg_kernel(g_hbm, srcs_hbm, dsts_hbm, zeros_hbm, out_hbm,
                src_v, dst_v, *rest):
    bufs = rest[:NBUF]
    agg_sh = rest[NBUF]
    sems = rest[NBUF + 1:]
    c = lax.axis_index("c")
    s = lax.axis_index("s")
    pltpu.sync_copy(zeros_hbm.at[pl.ds(s * STRIPE, STRIPE)],
                    agg_sh.at[pl.ds(s * STRIPE, STRIPE)])
    plsc.subcore_barrier()

    def fire(ch, b):
        for h in range(SUB):
            pltpu.async_copy(
                g_hbm.at[src_v.at[pl.ds(ch * CH + h * SUBR, SUBR)]],
                bufs[b].at[pl.ds(h * SUBR, SUBR)], sems[b])

    def drain(ch, b):
        for h in range(SUB):
            pltpu.make_async_copy(
                g_hbm.at[src_v.at[pl.ds(ch * CH + h * SUBR, SUBR)]],
                bufs[b].at[pl.ds(h * SUBR, SUBR)], sems[b]).wait()

    def run(base, nsec):
        for sec in range(nsec):
            off = base + sec * SCH
            pltpu.sync_copy(srcs_hbm.at[pl.ds(off * CH, SCH * CH)], src_v)
            pltpu.sync_copy(dsts_hbm.at[pl.ds(off, SCH)], dst_v)
            for b in range(NBUF):
                fire(b, b)

            def body(i, carry):
                for b in range(NBUF):
                    ch = i * NBUF + b
                    drain(ch, b)
                    pltpu.sync_copy(bufs[b], agg_sh.at[dst_v.at[ch]], add=True)

                    @pl.when(ch + NBUF < SCH)
                    def _fire():
                        fire(ch + NBUF, b)
                return carry

            lax.fori_loop(0, SCH // NBUF, body, 0)

    @pl.when(c == 0)
    def _fast_core():
        run(s * NCH0, NCH0 // SCH)

    @pl.when(c == 1)
    def _slow_core():
        run(BASE1 + s * NCH1, NCH1 // SCH)

    plsc.subcore_barrier()
    pltpu.sync_copy(agg_sh.at[pl.ds(s * STRIPE, STRIPE)],
                    out_hbm.at[c, pl.ds(s * STRIPE, STRIPE)])


BM = 2000


def _mm_scale_body(deg_ref, x_ref, w_ref, o_ref):
    d = lax.rsqrt(deg_ref[0] + deg_ref[1] + 1.0)[:, 0:1]
    h = jnp.dot(x_ref[...], w_ref[...], preferred_element_type=jnp.float32)
    o_ref[...] = h * d


def _layer_body(deg_ref, agg_ref, g_ref, w_ref, b_ref, o_ref):
    d = lax.rsqrt(deg_ref[0] + deg_ref[1] + 1.0)[:, 0:1]
    e = jnp.maximum(d * (agg_ref[0] + agg_ref[1] + g_ref[...]) + b_ref[...], 0.0)
    o_ref[...] = jnp.dot(e, w_ref[...], preferred_element_type=jnp.float32) * d


def _final_body(deg_ref, agg_ref, g_ref, w_ref, b_ref, bfc_ref, o_ref):
    d = lax.rsqrt(deg_ref[0] + deg_ref[1] + 1.0)[:, 0:1]
    e = jnp.maximum(d * (agg_ref[0] + agg_ref[1] + g_ref[...]) + b_ref[...], 0.0)
    o_ref[...] = jnp.dot(e, w_ref[...], preferred_element_type=jnp.float32) + bfc_ref[...]


_deg_spec = pl.BlockSpec((NC, BM, DEGW), lambda i: (0, i, 0))
_row_spec = pl.BlockSpec((BM, D), lambda i: (i, 0))
_agg_spec = pl.BlockSpec((NC, BM, D), lambda i: (0, i, 0))
_w_spec = pl.BlockSpec((D, D), lambda i: (0, 0))
_b_spec = pl.BlockSpec((1, D), lambda i: (0, 0))

_mm_scale = pl.pallas_call(
    _mm_scale_body,
    grid=(N // BM,),
    in_specs=[_deg_spec, _row_spec, _w_spec],
    out_specs=_row_spec,
    out_shape=jax.ShapeDtypeStruct((N, D), jnp.float32),
)

_layer = pl.pallas_call(
    _layer_body,
    grid=(N // BM,),
    in_specs=[_deg_spec, _agg_spec, _row_spec, _w_spec, _b_spec],
    out_specs=_row_spec,
    out_shape=jax.ShapeDtypeStruct((N, D), jnp.float32),
)

_final = pl.pallas_call(
    _final_body,
    grid=(N // BM,),
    in_specs=[_deg_spec, _agg_spec, _row_spec, _w_spec, _b_spec, _b_spec],
    out_specs=_row_spec,
    out_shape=jax.ShapeDtypeStruct((N, D), jnp.float32),
)


def kernel(x, edge_index, W1, b1, W2, b2, Wfc, bfc):
    src = edge_index[0].astype(jnp.int32)
    dst = edge_index[1].astype(jnp.int32)
    srcs = jnp.pad(src, (0, CAP - E))
    dsts = jnp.pad(dst, (0, CAP - E), constant_values=DUMMY).reshape(TCH, CH)
    dsts_d = dsts.reshape(NT, NCH, CH)

    zeros_d = jnp.zeros((NROWS, D), jnp.float32)
    ones_w = jnp.ones((CH, DEGW), jnp.float32)

    b1r = b1.reshape(1, D)
    b2r = b2.reshape(1, D)
    wfc_p = jnp.pad(Wfc, ((0, 0), (0, D - ACT)))
    bfc_p = jnp.pad(bfc, (0, D - ACT)).reshape(1, D)

    deg = _deg_kernel(dsts_d, ones_w, zeros_d)
    g1 = _mm_scale(deg, x, W1)
    a1 = _agg_kernel(g1, srcs, dsts, zeros_d)
    g2 = _layer(deg, a1, g1, W2, b1r)
    a2 = _agg_kernel(g2, srcs, dsts, zeros_d)
    q_pad = _final(deg, a2, g2, wfc_p, b2r, bfc_p)
    return q_pad[:, :ACT]


# speedup vs baseline: 3.1674x; 3.1674x over previous
import functools

import jax
import jax.numpy as jnp
from jax import lax
from jax.experimental import pallas as pl
from jax.experimental.pallas import tpu as pltpu
from jax.experimental.pallas import tpu_sc as plsc

N = 10000
E = 320000
D = 128
ACT = 18
NC = 2
NS = 16
NT = NC * NS
CH = 128
NCH = 80
TCH = NT * NCH
CAP = TCH * CH
SCH = 40
NCH0 = 80
NCH1 = 80
BASE1 = NS * NCH0
NROWS = 10112
STRIPE = NROWS // NS
DUMMY = 10008
DEGW = 128

NBUF = 2
SUB = 4
SUBR = CH // SUB

_mesh = plsc.VectorSubcoreMesh(core_axis_name="c", subcore_axis_name="s")


@functools.partial(
    pl.kernel,
    out_type=jax.ShapeDtypeStruct((NC, NROWS, DEGW), jnp.float32),
    mesh=_mesh,
    scratch_types=[
        pltpu.VMEM((NCH, CH), jnp.int32),
        pltpu.VMEM((CH, DEGW), jnp.float32),
        pltpu.VMEM_SHARED((NROWS, DEGW), jnp.float32),
    ],
)
def _deg_kernel(dsts_hbm, ones_hbm, zeros_hbm, out_hbm, dst_v, ones_v, deg_sh):
    c = lax.axis_index("c")
    s = lax.axis_index("s")
    wid = c * NS + s
    pltpu.sync_copy(dsts_hbm.at[wid], dst_v)
    pltpu.sync_copy(ones_hbm, ones_v)
    pltpu.sync_copy(zeros_hbm.at[pl.ds(s * STRIPE, STRIPE)],
                    deg_sh.at[pl.ds(s * STRIPE, STRIPE)])
    plsc.subcore_barrier()

    def body(i, carry):
        pltpu.sync_copy(ones_v, deg_sh.at[dst_v.at[i]], add=True)
        return carry

    lax.fori_loop(0, NCH, body, 0)
    plsc.subcore_barrier()
    pltpu.sync_copy(deg_sh.at[pl.ds(s * STRIPE, STRIPE)],
                    out_hbm.at[c, pl.ds(s * STRIPE, STRIPE)])


@functools.partial(
    pl.kernel,
    out_type=jax.ShapeDtypeStruct((NC, NROWS, D), jnp.float32),
    mesh=_mesh,
    scratch_types=[
        pltpu.VMEM((SCH * CH,), jnp.int32),
        pltpu.VMEM((SCH, CH), jnp.int32),
        *[pltpu.VMEM((CH, D), jnp.float32) for _ in range(NBUF)],
        pltpu.VMEM_SHARED((NROWS, D), jnp.float32),
        *[pltpu.SemaphoreType.DMA for _ in range(NBUF)],
    ],
)
def _agg_kernel(g_hbm, srcs_hbm, dsts_hbm, zeros_hbm, out_hbm,
                src_v, dst_v, *rest):
    bufs = rest[:NBUF]
    agg_sh = rest[NBUF]
    sems = rest[NBUF + 1:]
    c = lax.axis_index("c")
    s = lax.axis_index("s")
    pltpu.sync_copy(zeros_hbm.at[pl.ds(s * STRIPE, STRIPE)],
                    agg_sh.at[pl.ds(s * STRIPE, STRIPE)])
    plsc.subcore_barrier()

    def fire(ch, b):
        for h in range(SUB):
            pltpu.async_copy(
                g_hbm.at[src_v.at[pl.ds(ch * CH + h * SUBR, SUBR)]],
                bufs[b].at[pl.ds(h * SUBR, SUBR)], sems[b])

    def drain(ch, b):
        for h in range(SUB):
            pltpu.make_async_copy(
                g_hbm.at[src_v.at[pl.ds(ch * CH + h * SUBR, SUBR)]],
                bufs[b].at[pl.ds(h * SUBR, SUBR)], sems[b]).wait()

    def run(base, nsec):
        for sec in range(nsec):
            off = base + sec * SCH
            pltpu.sync_copy(srcs_hbm.at[pl.ds(off * CH, SCH * CH)], src_v)
            pltpu.sync_copy(dsts_hbm.at[pl.ds(off, SCH)], dst_v)
            for b in range(NBUF):
                fire(b, b)

            def body(i, carry):
                for b in range(NBUF):
                    ch = i * NBUF + b
                    drain(ch, b)
                    pltpu.sync_copy(bufs[b], agg_sh.at[dst_v.at[ch]], add=True)

                    @pl.when(ch + NBUF < SCH)
                    def _fire():
                        fire(ch + NBUF, b)
                return carry

            lax.fori_loop(0, SCH // NBUF, body, 0)

    @pl.when(c == 0)
    def _fast_core():
        run(s * NCH0, NCH0 // SCH)

    @pl.when(c == 1)
    def _slow_core():
        run(BASE1 + s * NCH1, NCH1 // SCH)

    plsc.subcore_barrier()
    pltpu.sync_copy(agg_sh.at[pl.ds(s * STRIPE, STRIPE)],
                    out_hbm.at[c, pl.ds(s * STRIPE, STRIPE)])


BM = 2000


def _mm_scale_body(deg_ref, x_ref, w_ref, o_ref):
    d = lax.rsqrt(deg_ref[0] + deg_ref[1] + 1.0)[:, 0:1]
    h = jnp.dot(x_ref[...], w_ref[...], preferred_element_type=jnp.float32)
    o_ref[...] = h * d


def _layer_body(deg_ref, agg_ref, g_ref, w_ref, b_ref, o_ref):
    d = lax.rsqrt(deg_ref[0] + deg_ref[1] + 1.0)[:, 0:1]
    e = jnp.maximum(d * (agg_ref[0] + agg_ref[1] + g_ref[...]) + b_ref[...], 0.0)
    o_ref[...] = jnp.dot(e, w_ref[...], preferred_element_type=jnp.float32) * d


def _final_body(deg_ref, agg_ref, g_ref, w_ref, b_ref, bfc_ref, o_ref):
    d = lax.rsqrt(deg_ref[0] + deg_ref[1] + 1.0)[:, 0:1]
    e = jnp.maximum(d * (agg_ref[0] + agg_ref[1] + g_ref[...]) + b_ref[...], 0.0)
    o_ref[...] = jnp.dot(e, w_ref[...], preferred_element_type=jnp.float32) + bfc_ref[...]


_deg_spec = pl.BlockSpec((NC, BM, DEGW), lambda i: (0, i, 0))
_row_spec = pl.BlockSpec((BM, D), lambda i: (i, 0))
_agg_spec = pl.BlockSpec((NC, BM, D), lambda i: (0, i, 0))
_w_spec = pl.BlockSpec((D, D), lambda i: (0, 0))
_b_spec = pl.BlockSpec((1, D), lambda i: (0, 0))

_mm_scale = pl.pallas_call(
    _mm_scale_body,
    grid=(N // BM,),
    in_specs=[_deg_spec, _row_spec, _w_spec],
    out_specs=_row_spec,
    out_shape=jax.ShapeDtypeStruct((N, D), jnp.float32),
)

_layer = pl.pallas_call(
    _layer_body,
    grid=(N // BM,),
    in_specs=[_deg_spec, _agg_spec, _row_spec, _w_spec, _b_spec],
    out_specs=_row_spec,
    out_shape=jax.ShapeDtypeStruct((N, D), jnp.float32),
)

_final = pl.pallas_call(
    _final_body,
    grid=(N // BM,),
    in_specs=[_deg_spec, _agg_spec, _row_spec, _w_spec, _b_spec, _b_spec],
    out_specs=_row_spec,
    out_shape=jax.ShapeDtypeStruct((N, D), jnp.float32),
)


def kernel(x, edge_index, W1, b1, W2, b2, Wfc, bfc):
    src = edge_index[0].astype(jnp.int32)
    dst = edge_index[1].astype(jnp.int32)
    srcs = jnp.concatenate([src, jnp.arange(CAP - E, dtype=jnp.int32)])
    dsts = jnp.pad(dst, (0, CAP - E), constant_values=DUMMY).reshape(TCH, CH)
    dsts_d = dsts.reshape(NT, NCH, CH)

    zeros_d = jnp.zeros((NROWS, D), jnp.float32)
    ones_w = jnp.ones((CH, DEGW), jnp.float32)

    b1r = b1.reshape(1, D)
    b2r = b2.reshape(1, D)
    wfc_p = jnp.pad(Wfc, ((0, 0), (0, D - ACT)))
    bfc_p = jnp.pad(bfc, (0, D - ACT)).reshape(1, D)

    deg = _deg_kernel(dsts_d, ones_w, zeros_d)
    g1 = _mm_scale(deg, x, W1)
    a1 = _agg_kernel(g1, srcs, dsts, zeros_d)
    g2 = _layer(deg, a1, g1, W2, b1r)
    a2 = _agg_kernel(g2, srcs, dsts, zeros_d)
    q_pad = _final(deg, a2, g2, wfc_p, b2r, bfc_p)
    return q_pad[:, :ACT]


# speedup vs baseline: 3.1687x; 1.0004x over previous
import functools

import jax
import jax.numpy as jnp
from jax import lax
from jax.experimental import pallas as pl
from jax.experimental.pallas import tpu as pltpu
from jax.experimental.pallas import tpu_sc as plsc

N = 10000
E = 320000
D = 128
ACT = 18
NC = 2
NS = 16
NT = NC * NS
CH = 128
NCH = 80
TCH = NT * NCH
CAP = TCH * CH
SCH = 40
NCH0 = 80
NCH1 = 80
BASE1 = NS * NCH0
NROWS = 10112
STRIPE = NROWS // NS
DUMMY = 10008
DEGW = 128

NBUF = 2
SUB = 1
SUBR = CH // SUB

_mesh = plsc.VectorSubcoreMesh(core_axis_name="c", subcore_axis_name="s")


@functools.partial(
    pl.kernel,
    out_type=jax.ShapeDtypeStruct((NC, NROWS, DEGW), jnp.float32),
    mesh=_mesh,
    scratch_types=[
        pltpu.VMEM((NCH, CH), jnp.int32),
        pltpu.VMEM((CH, DEGW), jnp.float32),
        pltpu.VMEM_SHARED((NROWS, DEGW), jnp.float32),
    ],
)
def _deg_kernel(dsts_hbm, ones_hbm, zeros_hbm, out_hbm, dst_v, ones_v, deg_sh):
    c = lax.axis_index("c")
    s = lax.axis_index("s")
    wid = c * NS + s
    pltpu.sync_copy(dsts_hbm.at[wid], dst_v)
    pltpu.sync_copy(ones_hbm, ones_v)
    pltpu.sync_copy(zeros_hbm.at[pl.ds(s * STRIPE, STRIPE)],
                    deg_sh.at[pl.ds(s * STRIPE, STRIPE)])
    plsc.subcore_barrier()

    def body(i, carry):
        pltpu.sync_copy(ones_v, deg_sh.at[dst_v.at[i]], add=True)
        return carry

    lax.fori_loop(0, NCH, body, 0)
    plsc.subcore_barrier()
    pltpu.sync_copy(deg_sh.at[pl.ds(s * STRIPE, STRIPE)],
                    out_hbm.at[c, pl.ds(s * STRIPE, STRIPE)])


@functools.partial(
    pl.kernel,
    out_type=jax.ShapeDtypeStruct((NC, NROWS, D), jnp.float32),
    mesh=_mesh,
    scratch_types=[
        pltpu.VMEM((SCH * CH,), jnp.int32),
        pltpu.VMEM((SCH, CH), jnp.int32),
        *[pltpu.VMEM((CH, D), jnp.float32) for _ in range(NBUF)],
        pltpu.VMEM_SHARED((NROWS, D), jnp.float32),
        *[pltpu.SemaphoreType.DMA for _ in range(NBUF)],
    ],
)
def _agg_kernel(g_hbm, srcs_hbm, dsts_hbm, zeros_hbm, out_hbm,
                src_v, dst_v, *rest):
    bufs = rest[:NBUF]
    agg_sh = rest[NBUF]
    sems = rest[NBUF + 1:]
    c = lax.axis_index("c")
    s = lax.axis_index("s")
    pltpu.sync_copy(zeros_hbm.at[pl.ds(s * STRIPE, STRIPE)],
                    agg_sh.at[pl.ds(s * STRIPE, STRIPE)])
    plsc.subcore_barrier()

    def fire(ch, b):
        for h in range(SUB):
            pltpu.async_copy(
                g_hbm.at[src_v.at[pl.ds(ch * CH + h * SUBR, SUBR)]],
                bufs[b].at[pl.ds(h * SUBR, SUBR)], sems[b])

    def drain(ch, b):
        for h in range(SUB):
            pltpu.make_async_copy(
                g_hbm.at[src_v.at[pl.ds(ch * CH + h * SUBR, SUBR)]],
                bufs[b].at[pl.ds(h * SUBR, SUBR)], sems[b]).wait()

    def run(base, nsec):
        for sec in range(nsec):
            off = base + sec * SCH
            pltpu.sync_copy(srcs_hbm.at[pl.ds(off * CH, SCH * CH)], src_v)
            pltpu.sync_copy(dsts_hbm.at[pl.ds(off, SCH)], dst_v)
            for b in range(NBUF):
                fire(b, b)

            def body(i, carry):
                for b in range(NBUF):
                    ch = i * NBUF + b
                    drain(ch, b)
                    pltpu.sync_copy(bufs[b], agg_sh.at[dst_v.at[ch]], add=True)

                    @pl.when(ch + NBUF < SCH)
                    def _fire():
                        fire(ch + NBUF, b)
                return carry

            lax.fori_loop(0, SCH // NBUF, body, 0)

    @pl.when(c == 0)
    def _fast_core():
        run(s * NCH0, NCH0 // SCH)

    @pl.when(c == 1)
    def _slow_core():
        run(BASE1 + s * NCH1, NCH1 // SCH)

    plsc.subcore_barrier()
    pltpu.sync_copy(agg_sh.at[pl.ds(s * STRIPE, STRIPE)],
                    out_hbm.at[c, pl.ds(s * STRIPE, STRIPE)])


BM = 2000


def _mm_scale_body(deg_ref, x_ref, w_ref, o_ref):
    d = lax.rsqrt(deg_ref[0] + deg_ref[1] + 1.0)[:, 0:1]
    h = jnp.dot(x_ref[...], w_ref[...], preferred_element_type=jnp.float32)
    o_ref[...] = h * d


def _layer_body(deg_ref, agg_ref, g_ref, w_ref, b_ref, o_ref):
    d = lax.rsqrt(deg_ref[0] + deg_ref[1] + 1.0)[:, 0:1]
    e = jnp.maximum(d * (agg_ref[0] + agg_ref[1] + g_ref[...]) + b_ref[...], 0.0)
    o_ref[...] = jnp.dot(e, w_ref[...], preferred_element_type=jnp.float32) * d


def _final_body(deg_ref, agg_ref, g_ref, w_ref, b_ref, bfc_ref, o_ref):
    d = lax.rsqrt(deg_ref[0] + deg_ref[1] + 1.0)[:, 0:1]
    e = jnp.maximum(d * (agg_ref[0] + agg_ref[1] + g_ref[...]) + b_ref[...], 0.0)
    o_ref[...] = jnp.dot(e, w_ref[...], preferred_element_type=jnp.float32) + bfc_ref[...]


_deg_spec = pl.BlockSpec((NC, BM, DEGW), lambda i: (0, i, 0))
_row_spec = pl.BlockSpec((BM, D), lambda i: (i, 0))
_agg_spec = pl.BlockSpec((NC, BM, D), lambda i: (0, i, 0))
_w_spec = pl.BlockSpec((D, D), lambda i: (0, 0))
_b_spec = pl.BlockSpec((1, D), lambda i: (0, 0))

_mm_scale = pl.pallas_call(
    _mm_scale_body,
    grid=(N // BM,),
    in_specs=[_deg_spec, _row_spec, _w_spec],
    out_specs=_row_spec,
    out_shape=jax.ShapeDtypeStruct((N, D), jnp.float32),
)

_layer = pl.pallas_call(
    _layer_body,
    grid=(N // BM,),
    in_specs=[_deg_spec, _agg_spec, _row_spec, _w_spec, _b_spec],
    out_specs=_row_spec,
    out_shape=jax.ShapeDtypeStruct((N, D), jnp.float32),
)

_final = pl.pallas_call(
    _final_body,
    grid=(N // BM,),
    in_specs=[_deg_spec, _agg_spec, _row_spec, _w_spec, _b_spec, _b_spec],
    out_specs=_row_spec,
    out_shape=jax.ShapeDtypeStruct((N, D), jnp.float32),
)


def kernel(x, edge_index, W1, b1, W2, b2, Wfc, bfc):
    src = edge_index[0].astype(jnp.int32)
    dst = edge_index[1].astype(jnp.int32)
    srcs = jnp.concatenate([src, jnp.arange(CAP - E, dtype=jnp.int32)])
    dsts = jnp.pad(dst, (0, CAP - E), constant_values=DUMMY).reshape(TCH, CH)
    dsts_d = dsts.reshape(NT, NCH, CH)

    zeros_d = jnp.zeros((NROWS, D), jnp.float32)
    ones_w = jnp.ones((CH, DEGW), jnp.float32)

    b1r = b1.reshape(1, D)
    b2r = b2.reshape(1, D)
    wfc_p = jnp.pad(Wfc, ((0, 0), (0, D - ACT)))
    bfc_p = jnp.pad(bfc, (0, D - ACT)).reshape(1, D)

    deg = _deg_kernel(dsts_d, ones_w, zeros_d)
    g1 = _mm_scale(deg, x, W1)
    a1 = _agg_kernel(g1, srcs, dsts, zeros_d)
    g2 = _layer(deg, a1, g1, W2, b1r)
    a2 = _agg_kernel(g2, srcs, dsts, zeros_d)
    q_pad = _final(deg, a2, g2, wfc_p, b2r, bfc_p)
    return q_pad[:, :ACT]
